# baseline (speedup 1.0000x reference)
import functools

import jax
import jax.numpy as jnp
from jax import lax
from jax.experimental import pallas as pl
from jax.experimental.pallas import tpu as pltpu
from jax.experimental.pallas import tpu_sc as plsc

_BATCH = 16384
_FIELDS = 26
_DIM = 64
_B = _BATCH * _FIELDS

_NC = 2
_NS = 16
_NW = _NC * _NS
_BPW = _B // _NW
_CHUNK = 416
_NCHUNK = _BPW // _CHUNK
_K = 4

_mesh = plsc.VectorSubcoreMesh(core_axis_name="c", subcore_axis_name="s")


@functools.partial(
    pl.kernel,
    out_type=jax.ShapeDtypeStruct((_B, _DIM), jnp.bfloat16),
    mesh=_mesh,
    scratch_types=[
        pltpu.VMEM((_BPW,), jnp.int32),
        [pltpu.VMEM((_CHUNK, _DIM), jnp.bfloat16) for _ in range(_K)],
        [pltpu.SemaphoreType.DMA for _ in range(_K)],
    ],
    compiler_params=pltpu.CompilerParams(use_tc_tiling_on_sc=False),
)
def _sc_gather(table_hbm, idx_hbm, out_hbm, idx_v, bufs, sems):
    wid = lax.axis_index("s") * _NC + lax.axis_index("c")
    base = wid * _BPW
    pltpu.sync_copy(idx_hbm.at[pl.ds(base, _BPW)], idx_v)
    for b in range(_K):
        pltpu.async_copy(table_hbm.at[idx_v.at[pl.ds(b * _CHUNK, _CHUNK)]],
                         bufs[b], sems[b])

    @pl.loop(0, _NCHUNK, step=_K)
    def _group(g):
        for b in range(_K):
            i = g + b
            pltpu.make_async_copy(
                table_hbm.at[idx_v.at[pl.ds(i * _CHUNK, _CHUNK)]],
                bufs[b], sems[b]).wait()
            pltpu.sync_copy(
                bufs[b],
                out_hbm.at[pl.ds(base + i * _CHUNK, _CHUNK)])

            @pl.when(i + _K < _NCHUNK)
            def _refill():
                pltpu.async_copy(
                    table_hbm.at[idx_v.at[pl.ds((i + _K) * _CHUNK, _CHUNK)]],
                    bufs[b], sems[b])


def kernel(x, weight):
    idx = jnp.maximum(x.reshape(_B).astype(jnp.int32), 0)
    out = _sc_gather(weight.astype(jnp.bfloat16), idx)
    return out.reshape(_BATCH, _FIELDS, _DIM).astype(jnp.float32)


# speedup vs baseline: 1.4942x; 1.4942x over previous
import functools

import jax
import jax.numpy as jnp
from jax import lax
from jax.experimental import pallas as pl
from jax.experimental.pallas import tpu as pltpu
from jax.experimental.pallas import tpu_sc as plsc

_BATCH = 16384
_FIELDS = 26
_DIM = 64
_B = _BATCH * _FIELDS

_NC = 2
_NS = 16
_NW = _NC * _NS
_BPW = _B // _NW
_CHUNK = 416
_NCHUNK = _BPW // _CHUNK
_K = 4

_mesh = plsc.VectorSubcoreMesh(core_axis_name="c", subcore_axis_name="s")


@functools.partial(
    pl.kernel,
    out_type=jax.ShapeDtypeStruct((_B, _DIM), jnp.float32),
    mesh=_mesh,
    scratch_types=[
        pltpu.VMEM((_BPW,), jnp.int32),
        [pltpu.VMEM((_CHUNK, _DIM), jnp.float32) for _ in range(_K)],
        [pltpu.SemaphoreType.DMA for _ in range(_K)],
    ],
    compiler_params=pltpu.CompilerParams(use_tc_tiling_on_sc=False),
)
def _sc_gather(table_hbm, idx_hbm, out_hbm, idx_v, bufs, sems):
    wid = lax.axis_index("s") * _NC + lax.axis_index("c")
    base = wid * _BPW
    pltpu.sync_copy(idx_hbm.at[pl.ds(base, _BPW)], idx_v)
    for b in range(_K):
        pltpu.async_copy(table_hbm.at[idx_v.at[pl.ds(b * _CHUNK, _CHUNK)]],
                         bufs[b], sems[b])

    @pl.loop(0, _NCHUNK, step=_K)
    def _group(g):
        for b in range(_K):
            i = g + b
            pltpu.make_async_copy(
                table_hbm.at[idx_v.at[pl.ds(i * _CHUNK, _CHUNK)]],
                bufs[b], sems[b]).wait()
            pltpu.sync_copy(
                bufs[b],
                out_hbm.at[pl.ds(base + i * _CHUNK, _CHUNK)])

            @pl.when(i + _K < _NCHUNK)
            def _refill():
                pltpu.async_copy(
                    table_hbm.at[idx_v.at[pl.ds((i + _K) * _CHUNK, _CHUNK)]],
                    bufs[b], sems[b])


def kernel(x, weight):
    idx = jnp.maximum(x.reshape(_B).astype(jnp.int32), 0)
    out = _sc_gather(weight, idx)
    return out.reshape(_BATCH, _FIELDS, _DIM)
